# baseline (speedup 1.0000x reference)
import functools

import jax
import jax.numpy as jnp
import numpy as np
from jax import lax
from jax.experimental import pallas as pl
from jax.experimental.pallas import tpu as pltpu
from jax.experimental.pallas import tpu_sc as plsc

B = 16384
F = 128
NC = 2
NS = 16
NW = NC * NS
BPW = B // NW
CH = 128
NCHUNK = BPW // CH

_GATHER_DNUMS = lax.GatherDimensionNumbers(
    offset_dims=(), collapsed_slice_dims=(0,), start_index_map=(0,))


def _permute(v, idx):
    return lax.gather(v, idx[:, None], _GATHER_DNUMS, (1,),
                      mode=lax.GatherScatterMode.PROMISE_IN_BOUNDS)


def _dot_rows(t_buf, i_buf, out_v, out_base, n):
    lane = lax.iota(jnp.int32, 16)

    def body(b, vec):
        acc = t_buf[b, pl.ds(0, 16)] * i_buf[b, pl.ds(0, 16)]
        for f in range(1, F // 16):
            acc = acc + t_buf[b, pl.ds(f * 16, 16)] * i_buf[b, pl.ds(f * 16, 16)]
        for sh in (8, 4, 2, 1):
            acc = acc + _permute(acc, jnp.bitwise_xor(lane, sh))
        k = b & 15
        vec = jnp.where(lane == k, acc, vec)

        @pl.when(k == 15)
        def _():
            out_v[pl.ds(out_base + b - 15, 16)] = vec

        return vec

    lax.fori_loop(0, n, body, jnp.zeros((16,), jnp.float32), unroll=8)


_mesh = plsc.VectorSubcoreMesh(core_axis_name="c", subcore_axis_name="s")


@functools.partial(
    pl.kernel,
    out_type=jax.ShapeDtypeStruct((B,), jnp.float32),
    mesh=_mesh,
    scratch_types=[
        pltpu.VMEM((BPW,), jnp.int32),
        pltpu.VMEM((BPW,), jnp.int32),
        pltpu.VMEM((2, CH, F), jnp.float32),
        pltpu.VMEM((2, CH, F), jnp.float32),
        pltpu.VMEM((BPW,), jnp.float32),
        pltpu.SemaphoreType.DMA,
        pltpu.SemaphoreType.DMA,
        pltpu.SemaphoreType.DMA,
        pltpu.SemaphoreType.DMA,
    ],
)
def _mf_kernel(time_hbm, item_hbm, tf_hbm, if_hbm, out_hbm,
               tidx_v, iidx_v, t_buf, i_buf, out_v,
               sem_t0, sem_i0, sem_t1, sem_i1):
    wid = lax.axis_index("s") * NC + lax.axis_index("c")
    base = wid * BPW
    pltpu.sync_copy(time_hbm.at[pl.ds(base, BPW)], tidx_v)
    pltpu.sync_copy(item_hbm.at[pl.ds(base, BPW)], iidx_v)

    def start(c, slot, sem_t, sem_i):
        pltpu.async_copy(
            tf_hbm.at[tidx_v.at[pl.ds(c * CH, CH)]], t_buf.at[slot], sem_t)
        pltpu.async_copy(
            if_hbm.at[iidx_v.at[pl.ds(c * CH, CH)]], i_buf.at[slot], sem_i)

    def wait(slot, sem_t, sem_i):
        pltpu.make_async_copy(tf_hbm.at[pl.ds(0, CH)], t_buf.at[slot],
                              sem_t).wait()
        pltpu.make_async_copy(if_hbm.at[pl.ds(0, CH)], i_buf.at[slot],
                              sem_i).wait()

    start(0, 0, sem_t0, sem_i0)
    start(1, 1, sem_t1, sem_i1)

    def body(i, carry):
        c = 2 * i
        wait(0, sem_t0, sem_i0)
        _dot_rows(t_buf.at[0], i_buf.at[0], out_v, c * CH, CH)

        @pl.when(c + 2 < NCHUNK)
        def _():
            start(c + 2, 0, sem_t0, sem_i0)

        wait(1, sem_t1, sem_i1)
        _dot_rows(t_buf.at[1], i_buf.at[1], out_v, (c + 1) * CH, CH)

        @pl.when(c + 3 < NCHUNK)
        def _():
            start(c + 3, 1, sem_t1, sem_i1)
        return carry

    lax.fori_loop(0, NCHUNK // 2, body, 0)
    pltpu.sync_copy(out_v, out_hbm.at[pl.ds(base, BPW)])


def kernel(time, item, time_factor, item_factor, lag_factor):
    del lag_factor
    return _mf_kernel(time, item, time_factor, item_factor)


# speedup vs baseline: 1.0541x; 1.0541x over previous
import functools

import jax
import jax.numpy as jnp
import numpy as np
from jax import lax
from jax.experimental import pallas as pl
from jax.experimental.pallas import tpu as pltpu
from jax.experimental.pallas import tpu_sc as plsc

B = 16384
F = 128
NC = 2
NS = 16
NW = NC * NS
BPW = B // NW
CH = 128
NCHUNK = BPW // CH

_GATHER_DNUMS = lax.GatherDimensionNumbers(
    offset_dims=(), collapsed_slice_dims=(0,), start_index_map=(0,))


def _permute(v, idx):
    return lax.gather(v, idx[:, None], _GATHER_DNUMS, (1,),
                      mode=lax.GatherScatterMode.PROMISE_IN_BOUNDS)


def _dot_rows(t_buf, i_buf, out_v, out_base, n):
    lane = lax.iota(jnp.int32, 16)

    def make_row(g):
        def row(k, vec):
            return _one_row(g * 16 + k, vec)
        return row

    def _one_row(b, vec):
        acc = t_buf[b, pl.ds(0, 16)] * i_buf[b, pl.ds(0, 16)]
        for f in range(1, F // 16):
            acc = acc + t_buf[b, pl.ds(f * 16, 16)] * i_buf[b, pl.ds(f * 16, 16)]
        for sh in (8, 4, 2, 1):
            acc = acc + _permute(acc, jnp.bitwise_xor(lane, sh))
        return jnp.where(lane == (b & 15), acc, vec)

    def group(g, carry):
        vec = lax.fori_loop(0, 16, make_row(g),
                            jnp.zeros((16,), jnp.float32), unroll=4)
        out_v[pl.ds(out_base + g * 16, 16)] = vec
        return carry

    lax.fori_loop(0, n // 16, group, 0)


_mesh = plsc.VectorSubcoreMesh(core_axis_name="c", subcore_axis_name="s")


@functools.partial(
    pl.kernel,
    out_type=jax.ShapeDtypeStruct((B,), jnp.float32),
    mesh=_mesh,
    scratch_types=[
        pltpu.VMEM((BPW,), jnp.int32),
        pltpu.VMEM((BPW,), jnp.int32),
        pltpu.VMEM((2, CH, F), jnp.float32),
        pltpu.VMEM((2, CH, F), jnp.float32),
        pltpu.VMEM((BPW,), jnp.float32),
        pltpu.SemaphoreType.DMA,
        pltpu.SemaphoreType.DMA,
        pltpu.SemaphoreType.DMA,
        pltpu.SemaphoreType.DMA,
    ],
)
def _mf_kernel(time_hbm, item_hbm, tf_hbm, if_hbm, out_hbm,
               tidx_v, iidx_v, t_buf, i_buf, out_v,
               sem_t0, sem_i0, sem_t1, sem_i1):
    wid = lax.axis_index("s") * NC + lax.axis_index("c")
    base = wid * BPW
    pltpu.sync_copy(time_hbm.at[pl.ds(base, BPW)], tidx_v)
    pltpu.sync_copy(item_hbm.at[pl.ds(base, BPW)], iidx_v)

    def start(c, slot, sem_t, sem_i):
        pltpu.async_copy(
            tf_hbm.at[tidx_v.at[pl.ds(c * CH, CH)]], t_buf.at[slot], sem_t)
        pltpu.async_copy(
            if_hbm.at[iidx_v.at[pl.ds(c * CH, CH)]], i_buf.at[slot], sem_i)

    def wait(slot, sem_t, sem_i):
        pltpu.make_async_copy(tf_hbm.at[pl.ds(0, CH)], t_buf.at[slot],
                              sem_t).wait()
        pltpu.make_async_copy(if_hbm.at[pl.ds(0, CH)], i_buf.at[slot],
                              sem_i).wait()

    start(0, 0, sem_t0, sem_i0)
    start(1, 1, sem_t1, sem_i1)

    def body(i, carry):
        c = 2 * i
        wait(0, sem_t0, sem_i0)
        _dot_rows(t_buf.at[0], i_buf.at[0], out_v, c * CH, CH)

        @pl.when(c + 2 < NCHUNK)
        def _():
            start(c + 2, 0, sem_t0, sem_i0)

        wait(1, sem_t1, sem_i1)
        _dot_rows(t_buf.at[1], i_buf.at[1], out_v, (c + 1) * CH, CH)

        @pl.when(c + 3 < NCHUNK)
        def _():
            start(c + 3, 1, sem_t1, sem_i1)
        return carry

    lax.fori_loop(0, NCHUNK // 2, body, 0)
    pltpu.sync_copy(out_v, out_hbm.at[pl.ds(base, BPW)])


def kernel(time, item, time_factor, item_factor, lag_factor):
    del lag_factor
    return _mf_kernel(time, item, time_factor, item_factor)
